# initial kernel scaffold (unmeasured)
import jax
import jax.numpy as jnp
from jax import lax
from jax.experimental import pallas as pl
from jax.experimental.pallas import tpu as pltpu


def kernel(
    x,
):
    def body(*refs):
        pass

    out_shape = jax.ShapeDtypeStruct(..., jnp.float32)
    return pl.pallas_call(body, out_shape=out_shape)(...)


# baseline (device time: 4379907 ns/iter reference)
import jax
import jax.numpy as jnp
from jax import lax
from jax.experimental import pallas as pl
from jax.experimental.pallas import tpu as pltpu

N_Y = 4


def kernel(x):
    m_per, n = x.shape

    def body(x_ref, out_ref, copy_sem, send_sems, recv_sems):
        my_x = lax.axis_index("x")
        my_y = lax.axis_index("y")
        my_z = lax.axis_index("z")
        left = (my_y - 1 + N_Y) % N_Y
        right = (my_y + 1) % N_Y

        barrier_sem = pltpu.get_barrier_semaphore()
        for nbr in [left, right]:
            pl.semaphore_signal(
                barrier_sem,
                inc=1,
                device_id=(my_x, nbr, my_z),
                device_id_type=pl.DeviceIdType.MESH,
            )
        pl.semaphore_wait(barrier_sem, 2)

        local = pltpu.make_async_copy(
            x_ref, out_ref.at[pl.ds(my_y * m_per, m_per), :], copy_sem
        )
        local.start()
        local.wait()

        for h in range(N_Y - 1):
            origin = (my_y - h + N_Y) % N_Y
            rdma = pltpu.make_async_remote_copy(
                src_ref=out_ref.at[pl.ds(origin * m_per, m_per), :],
                dst_ref=out_ref.at[pl.ds(origin * m_per, m_per), :],
                send_sem=send_sems.at[h],
                recv_sem=recv_sems.at[h],
                device_id=(my_x, right, my_z),
                device_id_type=pl.DeviceIdType.MESH,
            )
            rdma.start()
            rdma.wait()

    return pl.pallas_call(
        body,
        out_shape=jax.ShapeDtypeStruct((N_Y * m_per, n), x.dtype),
        in_specs=[pl.BlockSpec(memory_space=pl.ANY)],
        out_specs=pl.BlockSpec(memory_space=pl.ANY),
        scratch_shapes=[
            pltpu.SemaphoreType.DMA,
            pltpu.SemaphoreType.DMA((N_Y - 1,)),
            pltpu.SemaphoreType.DMA((N_Y - 1,)),
        ],
        compiler_params=pltpu.CompilerParams(collective_id=0),
    )(x)


# device time: 2215621 ns/iter; 1.9768x vs baseline; 1.9768x over previous
import jax
import jax.numpy as jnp
from jax import lax
from jax.experimental import pallas as pl
from jax.experimental.pallas import tpu as pltpu

N_Y = 4
K = 4


def kernel(x):
    m_per, n = x.shape
    H = m_per // 2
    B = H // K

    def body(x_ref, out_ref, copy_sem, ssem_r, rsem_r, ssem_l, rsem_l,
             ssem_x, rsem_x):
        my_x = lax.axis_index("x")
        my_y = lax.axis_index("y")
        my_z = lax.axis_index("z")
        partner = (1 - my_x, my_y, my_z)
        right_dev = (my_x, my_y + 1, my_z)
        left_dev = (my_x, my_y - 1, my_z)
        has_left = my_y > 0
        has_right = my_y < N_Y - 1

        def my_block(c, b):
            c = jnp.clip(c, 0, N_Y - 1)
            return out_ref.at[pl.ds(c * m_per + my_x * H + b * B, B), :]

        def partner_block(c, b):
            c = jnp.clip(c, 0, N_Y - 1)
            return out_ref.at[pl.ds(c * m_per + (1 - my_x) * H + b * B, B), :]

        bar = pltpu.get_barrier_semaphore()
        pl.semaphore_signal(bar, inc=1, device_id=partner,
                            device_id_type=pl.DeviceIdType.MESH)

        @pl.when(has_left)
        def _():
            pl.semaphore_signal(bar, inc=1, device_id=left_dev,
                                device_id_type=pl.DeviceIdType.MESH)

        @pl.when(has_right)
        def _():
            pl.semaphore_signal(bar, inc=1, device_id=right_dev,
                                device_id_type=pl.DeviceIdType.MESH)

        both = jnp.logical_and(has_left, has_right)

        @pl.when(both)
        def _():
            pl.semaphore_wait(bar, 3)

        @pl.when(jnp.logical_not(both))
        def _():
            pl.semaphore_wait(bar, 2)

        local = pltpu.make_async_copy(
            x_ref, out_ref.at[pl.ds(my_y * m_per, m_per), :], copy_sem)
        local.start()

        for s in range(N_Y - 1):
            cr = my_y - s
            cl = my_y + s
            can_r = jnp.logical_and(cr >= 0, has_right)
            can_l = jnp.logical_and(cl <= N_Y - 1, has_left)
            for b in range(K):
                @pl.when(can_r)
                def _(s=s, b=b, cr=cr):
                    src = (x_ref.at[pl.ds(my_x * H + b * B, B), :]
                           if s == 0 else my_block(cr, b))
                    rd = pltpu.make_async_remote_copy(
                        src_ref=src, dst_ref=my_block(cr, b),
                        send_sem=ssem_r.at[s, b], recv_sem=rsem_r.at[s, b],
                        device_id=right_dev,
                        device_id_type=pl.DeviceIdType.MESH)
                    rd.start()

                @pl.when(can_l)
                def _(s=s, b=b, cl=cl):
                    src = (x_ref.at[pl.ds(my_x * H + b * B, B), :]
                           if s == 0 else my_block(cl, b))
                    rd = pltpu.make_async_remote_copy(
                        src_ref=src, dst_ref=my_block(cl, b),
                        send_sem=ssem_l.at[s, b], recv_sem=rsem_l.at[s, b],
                        device_id=left_dev,
                        device_id_type=pl.DeviceIdType.MESH)
                    rd.start()

            crl = my_y - 1 - s
            crr = my_y + 1 + s
            got_l = jnp.logical_and(crl >= 0, has_left)
            got_r = jnp.logical_and(crr <= N_Y - 1, has_right)
            for b in range(K):
                @pl.when(got_l)
                def _(s=s, b=b, crl=crl):
                    rd = pltpu.make_async_remote_copy(
                        src_ref=my_block(crl, b), dst_ref=my_block(crl, b),
                        send_sem=ssem_r.at[s, b], recv_sem=rsem_r.at[s, b],
                        device_id=left_dev,
                        device_id_type=pl.DeviceIdType.MESH)
                    rd.wait_recv()
                    xs = pltpu.make_async_remote_copy(
                        src_ref=my_block(crl, b), dst_ref=my_block(crl, b),
                        send_sem=ssem_x.at[s, 0, b],
                        recv_sem=rsem_x.at[s, 0, b],
                        device_id=partner,
                        device_id_type=pl.DeviceIdType.MESH)
                    xs.start()

                @pl.when(got_r)
                def _(s=s, b=b, crr=crr):
                    rd = pltpu.make_async_remote_copy(
                        src_ref=my_block(crr, b), dst_ref=my_block(crr, b),
                        send_sem=ssem_l.at[s, b], recv_sem=rsem_l.at[s, b],
                        device_id=right_dev,
                        device_id_type=pl.DeviceIdType.MESH)
                    rd.wait_recv()
                    xs = pltpu.make_async_remote_copy(
                        src_ref=my_block(crr, b), dst_ref=my_block(crr, b),
                        send_sem=ssem_x.at[s, 1, b],
                        recv_sem=rsem_x.at[s, 1, b],
                        device_id=partner,
                        device_id_type=pl.DeviceIdType.MESH)
                    xs.start()

        for s in range(N_Y - 1):
            crl = my_y - 1 - s
            crr = my_y + 1 + s
            got_l = jnp.logical_and(crl >= 0, has_left)
            got_r = jnp.logical_and(crr <= N_Y - 1, has_right)
            for b in range(K):
                @pl.when(got_l)
                def _(s=s, b=b, crl=crl):
                    xr = pltpu.make_async_remote_copy(
                        src_ref=my_block(crl, b),
                        dst_ref=partner_block(crl, b),
                        send_sem=ssem_x.at[s, 0, b],
                        recv_sem=rsem_x.at[s, 0, b],
                        device_id=partner,
                        device_id_type=pl.DeviceIdType.MESH)
                    xr.wait_recv()

                @pl.when(got_r)
                def _(s=s, b=b, crr=crr):
                    xr = pltpu.make_async_remote_copy(
                        src_ref=my_block(crr, b),
                        dst_ref=partner_block(crr, b),
                        send_sem=ssem_x.at[s, 1, b],
                        recv_sem=rsem_x.at[s, 1, b],
                        device_id=partner,
                        device_id_type=pl.DeviceIdType.MESH)
                    xr.wait_recv()

        for s in range(N_Y - 1):
            cr = my_y - s
            cl = my_y + s
            can_r = jnp.logical_and(cr >= 0, has_right)
            can_l = jnp.logical_and(cl <= N_Y - 1, has_left)
            crl = my_y - 1 - s
            crr = my_y + 1 + s
            got_l = jnp.logical_and(crl >= 0, has_left)
            got_r = jnp.logical_and(crr <= N_Y - 1, has_right)
            for b in range(K):
                @pl.when(can_r)
                def _(s=s, b=b, cr=cr):
                    src = (x_ref.at[pl.ds(my_x * H + b * B, B), :]
                           if s == 0 else my_block(cr, b))
                    rd = pltpu.make_async_remote_copy(
                        src_ref=src, dst_ref=my_block(cr, b),
                        send_sem=ssem_r.at[s, b], recv_sem=rsem_r.at[s, b],
                        device_id=right_dev,
                        device_id_type=pl.DeviceIdType.MESH)
                    rd.wait_send()

                @pl.when(can_l)
                def _(s=s, b=b, cl=cl):
                    src = (x_ref.at[pl.ds(my_x * H + b * B, B), :]
                           if s == 0 else my_block(cl, b))
                    rd = pltpu.make_async_remote_copy(
                        src_ref=src, dst_ref=my_block(cl, b),
                        send_sem=ssem_l.at[s, b], recv_sem=rsem_l.at[s, b],
                        device_id=left_dev,
                        device_id_type=pl.DeviceIdType.MESH)
                    rd.wait_send()

                @pl.when(got_l)
                def _(s=s, b=b, crl=crl):
                    xs = pltpu.make_async_remote_copy(
                        src_ref=my_block(crl, b), dst_ref=my_block(crl, b),
                        send_sem=ssem_x.at[s, 0, b],
                        recv_sem=rsem_x.at[s, 0, b],
                        device_id=partner,
                        device_id_type=pl.DeviceIdType.MESH)
                    xs.wait_send()

                @pl.when(got_r)
                def _(s=s, b=b, crr=crr):
                    xs = pltpu.make_async_remote_copy(
                        src_ref=my_block(crr, b), dst_ref=my_block(crr, b),
                        send_sem=ssem_x.at[s, 1, b],
                        recv_sem=rsem_x.at[s, 1, b],
                        device_id=partner,
                        device_id_type=pl.DeviceIdType.MESH)
                    xs.wait_send()

        local.wait()

    return pl.pallas_call(
        body,
        out_shape=jax.ShapeDtypeStruct((N_Y * m_per, n), x.dtype),
        in_specs=[pl.BlockSpec(memory_space=pl.ANY)],
        out_specs=pl.BlockSpec(memory_space=pl.ANY),
        scratch_shapes=[
            pltpu.SemaphoreType.DMA,
            pltpu.SemaphoreType.DMA((N_Y - 1, K)),
            pltpu.SemaphoreType.DMA((N_Y - 1, K)),
            pltpu.SemaphoreType.DMA((N_Y - 1, K)),
            pltpu.SemaphoreType.DMA((N_Y - 1, K)),
            pltpu.SemaphoreType.DMA((N_Y - 1, 2, K)),
            pltpu.SemaphoreType.DMA((N_Y - 1, 2, K)),
        ],
        compiler_params=pltpu.CompilerParams(collective_id=0),
    )(x)
